# Initial kernel scaffold; baseline (speedup 1.0000x reference)
import jax
import jax.numpy as jnp
from jax.experimental import pallas as pl


def kernel(x, table):
    raise NotImplementedError("write your pallas kernel here")


# speedup vs baseline: 1.7978x; 1.7978x over previous
import functools

import jax
import jax.numpy as jnp
from jax import lax
from jax.experimental import pallas as pl
from jax.experimental.pallas import tpu as pltpu
from jax.experimental.pallas import tpu_sc as plsc

D = 64
NC, NS = 2, 16
NW = NC * NS
CHUNK = 512


def _emb_body(idx_hbm, table_hbm, out_hbm, idx_v, rows_v, sem):
    b_total = idx_hbm.shape[0]
    b_per_w = b_total // NW
    nchunks = b_per_w // CHUNK
    wid = lax.axis_index("s") * NC + lax.axis_index("c")
    base = wid * b_per_w

    def step(i, carry):
        off = base + i * CHUNK
        pltpu.sync_copy(idx_hbm.at[pl.ds(off, CHUNK)], idx_v)
        pltpu.async_copy(table_hbm.at[idx_v], rows_v, sem).wait()
        pltpu.sync_copy(rows_v, out_hbm.at[pl.ds(off, CHUNK)])
        return carry

    lax.fori_loop(0, nchunks, step, 0)


def kernel(x, table):
    b, h = x.shape
    flat = x.reshape(b * h)
    mesh = plsc.VectorSubcoreMesh(core_axis_name="c", subcore_axis_name="s")
    run = pl.kernel(
        _emb_body,
        out_type=jax.ShapeDtypeStruct((b * h, D), jnp.float32),
        mesh=mesh,
        scratch_types=[
            pltpu.VMEM((CHUNK,), jnp.int32),
            pltpu.VMEM((CHUNK, D), jnp.float32),
            pltpu.SemaphoreType.DMA,
        ],
        compiler_params=pltpu.CompilerParams(use_tc_tiling_on_sc=False),
    )
    out = run(flat, table)
    return out.reshape(b, h, D)


# speedup vs baseline: 1.8765x; 1.0438x over previous
import functools

import jax
import jax.numpy as jnp
from jax import lax
from jax.experimental import pallas as pl
from jax.experimental.pallas import tpu as pltpu
from jax.experimental.pallas import tpu_sc as plsc

D = 64
NC, NS = 2, 16
NW = NC * NS
CHUNK = 256
NBUF = 4
K = NBUF // 2


def _emb_body(idx_hbm, table_hbm, out_hbm, idx_v, rows_v, sem_g, sem_s):
    b_total = idx_hbm.shape[0]
    b_per_w = b_total // NW
    nchunks = b_per_w // CHUNK
    ngroups = nchunks // NBUF
    wid = lax.axis_index("s") * NC + lax.axis_index("c")
    base = wid * b_per_w

    def group(g, carry):
        for b in range(NBUF):
            i = g * NBUF + b

            @pl.when(i >= NBUF)
            def _():
                pltpu.make_async_copy(
                    rows_v.at[b],
                    out_hbm.at[pl.ds(base + (i - NBUF) * CHUNK, CHUNK)],
                    sem_s.at[b],
                ).wait()

            pltpu.sync_copy(idx_hbm.at[pl.ds(base + i * CHUNK, CHUNK)],
                            idx_v.at[b])
            pltpu.async_copy(table_hbm.at[idx_v.at[b]], rows_v.at[b],
                             sem_g.at[b])

            j = i - K
            bj = (b - K) % NBUF

            @pl.when(j >= 0)
            def _():
                pltpu.make_async_copy(table_hbm.at[idx_v.at[bj]],
                                      rows_v.at[bj], sem_g.at[bj]).wait()
                pltpu.async_copy(
                    rows_v.at[bj],
                    out_hbm.at[pl.ds(base + j * CHUNK, CHUNK)],
                    sem_s.at[bj],
                )
        return carry

    lax.fori_loop(0, ngroups, group, 0)

    for t in range(K):
        j = nchunks - K + t
        bj = j % NBUF
        pltpu.make_async_copy(table_hbm.at[idx_v.at[bj]], rows_v.at[bj],
                              sem_g.at[bj]).wait()
        pltpu.async_copy(rows_v.at[bj],
                         out_hbm.at[pl.ds(base + j * CHUNK, CHUNK)],
                         sem_s.at[bj])
    for b in range(NBUF):
        j = nchunks - NBUF + b
        pltpu.make_async_copy(rows_v.at[b],
                              out_hbm.at[pl.ds(base + j * CHUNK, CHUNK)],
                              sem_s.at[b]).wait()


def kernel(x, table):
    b, h = x.shape
    flat = x.reshape(b * h)
    mesh = plsc.VectorSubcoreMesh(core_axis_name="c", subcore_axis_name="s")
    run = pl.kernel(
        _emb_body,
        out_type=jax.ShapeDtypeStruct((b * h, D), jnp.float32),
        mesh=mesh,
        scratch_types=[
            pltpu.VMEM((NBUF, CHUNK), jnp.int32),
            pltpu.VMEM((NBUF, CHUNK, D), jnp.float32),
            pltpu.SemaphoreType.DMA((NBUF,)),
            pltpu.SemaphoreType.DMA((NBUF,)),
        ],
        compiler_params=pltpu.CompilerParams(use_tc_tiling_on_sc=False),
    )
    out = run(flat, table)
    return out.reshape(b, h, D)
